# baseline (speedup 1.0000x reference)
from functools import partial

import jax
import jax.numpy as jnp
from jax.experimental import pallas as pl
from jax.experimental.pallas import tpu as pltpu


def _fused_mbstd_kernel(x_ref, o_ref, acc_ref, *, inv_d):
    k = pl.program_id(1)
    j = pl.program_id(2)

    @pl.when((k == 0) & (j == 0))
    def _init():
        acc_ref[...] = jnp.zeros_like(acc_ref)

    @pl.when(k == 0)
    def _copy_and_accumulate():
        xv = x_ref[...]
        o_ref[...] = xv
        x32 = xv.astype(jnp.float32)
        mean = jnp.mean(x32, axis=0)
        var = jnp.mean(jnp.square(x32 - mean), axis=0)
        std = jnp.sqrt(var + 1e-8)
        acc_ref[...] += jnp.sum(std, axis=(1, 2))[:, None] * inv_d

    @pl.when(k == 1)
    def _write_stat_channel():
        g, mt, hw_t, _ = o_ref.shape
        s = acc_ref[...][None, :, :, None]
        o_ref[:, :, :, 0:1] = jnp.broadcast_to(
            s, (g, mt, hw_t, 1)).astype(o_ref.dtype)


def kernel(x):
    b, C, h, w = x.shape
    G = min(b, 4)
    assert b % G == 0
    m = b // G
    HW = h * w
    D = C * HW

    budget = 8 * 1024 * 1024
    Mt, hw_t = m, HW
    while hw_t * C * G * Mt * x.dtype.itemsize > budget and Mt % 2 == 0 and Mt > 2:
        Mt //= 2
    while hw_t * C * G * Mt * x.dtype.itemsize > budget and hw_t % 2 == 0:
        hw_t //= 2
    n_m = m // Mt
    n_hw = HW // hw_t

    xg = jnp.transpose(x, (0, 2, 3, 1)).reshape(G, m, HW, C)
    y = pl.pallas_call(
        partial(_fused_mbstd_kernel, inv_d=1.0 / D),
        out_shape=jax.ShapeDtypeStruct((G, m, HW, C + 1), x.dtype),
        grid=(n_m, 2, n_hw),
        in_specs=[
            pl.BlockSpec((G, Mt, hw_t, C),
                         lambda i, k, j: (0, i, jnp.where(k == 0, j, n_hw - 1), 0)),
        ],
        out_specs=pl.BlockSpec((G, Mt, hw_t, C), lambda i, k, j: (0, i, j, k)),
        scratch_shapes=[pltpu.VMEM((Mt, 1), jnp.float32)],
        compiler_params=pltpu.CompilerParams(
            dimension_semantics=("parallel", "arbitrary", "arbitrary"),
            vmem_limit_bytes=40 * 1024 * 1024),
    )(xg)
    yt = y.reshape(b, h, w, C + 1)
    return jnp.transpose(yt, (0, 3, 1, 2))


# speedup vs baseline: 1.4186x; 1.4186x over previous
from functools import partial

import jax
import jax.numpy as jnp
from jax.experimental import pallas as pl
from jax.experimental.pallas import tpu as pltpu


def _fused_mbstd_kernel(x_ref, o_ref, acc_ref, *, inv_d):
    k = pl.program_id(1)
    j = pl.program_id(2)

    @pl.when((k == 0) & (j == 0))
    def _init():
        acc_ref[...] = jnp.zeros_like(acc_ref)

    @pl.when(k == 0)
    def _copy_and_accumulate():
        xv = x_ref[...]
        o_ref[...] = xv
        x32 = xv.astype(jnp.float32)
        mean = jnp.mean(x32, axis=0)
        var = jnp.mean(jnp.square(x32 - mean), axis=0)
        std = jnp.sqrt(var + 1e-8)
        acc_ref[...] += jnp.sum(std, axis=(1, 2))[:, None] * inv_d

    @pl.when(k == 1)
    def _write_stat_channel():
        g, mt, hw_t, _ = o_ref.shape
        s = acc_ref[...][None, :, :, None]
        o_ref[:, :, :, 0:1] = jnp.broadcast_to(
            s, (g, mt, hw_t, 1)).astype(o_ref.dtype)


def kernel(x):
    b, C, h, w = x.shape
    G = min(b, 4)
    assert b % G == 0
    m = b // G
    HW = h * w
    D = C * HW

    Mt = 8 if (m >= 16 and m % 8 == 0) else m
    n_m = m // Mt

    hw_t = HW
    while hw_t * C * G * Mt * x.dtype.itemsize > 8 * 1024 * 1024 and hw_t % 2 == 0:
        hw_t //= 2
    n_hw = HW // hw_t

    xg = jnp.transpose(x, (0, 2, 3, 1)).reshape(G, m, HW, C)
    y = pl.pallas_call(
        partial(_fused_mbstd_kernel, inv_d=1.0 / D),
        out_shape=jax.ShapeDtypeStruct((G, m, HW, C + 1), x.dtype),
        grid=(n_m, 2, n_hw),
        in_specs=[
            pl.BlockSpec((G, Mt, hw_t, C),
                         lambda i, k, j: (0, i, jnp.where(k == 0, j, n_hw - 1), 0)),
        ],
        out_specs=pl.BlockSpec((G, Mt, hw_t, C), lambda i, k, j: (0, i, j, k)),
        scratch_shapes=[pltpu.VMEM((Mt, 1), jnp.float32)],
        compiler_params=pltpu.CompilerParams(
            dimension_semantics=("parallel", "arbitrary", "arbitrary"),
            vmem_limit_bytes=40 * 1024 * 1024),
    )(xg)
    yt = y.reshape(b, h, w, C + 1)
    return jnp.transpose(yt, (0, 3, 1, 2))
